# baseline (speedup 1.0000x reference)
import functools

import jax
import jax.numpy as jnp
from jax import lax
from jax.experimental import pallas as pl
from jax.experimental.pallas import tpu as pltpu
from jax.experimental.pallas import tpu_sc as plsc

DIM = 64
PD = 128
HIST_PAD = 56
NC = 2
NS = 16
NW = NC * NS
CHUNK = 256
VBLK = 2048


def _transpose_body(wt_ref, out_ref):
    eye = jnp.eye(DIM, dtype=jnp.float32)
    t = jax.lax.dot_general(
        wt_ref[...], eye, (((0,), (0,)), ((), ())),
        precision=jax.lax.Precision.HIGHEST)
    out_ref[:, : DIM] = t


@functools.cache
def _make_transpose(V: int):
    grid = (V + VBLK - 1) // VBLK
    return pl.pallas_call(
        _transpose_body,
        grid=(grid,),
        in_specs=[pl.BlockSpec((DIM, VBLK), lambda j: (0, j))],
        out_specs=pl.BlockSpec((VBLK, PD), lambda j: (j, 0)),
        out_shape=jax.ShapeDtypeStruct((V, PD), jnp.float32),
    )


@functools.cache
def _make_gather(BP: int):
    b_per_w = BP // NW
    n_chunks = b_per_w // CHUNK
    assert n_chunks % 2 == 0
    mesh = plsc.VectorSubcoreMesh(core_axis_name="c", subcore_axis_name="s")

    @functools.partial(
        pl.kernel,
        mesh=mesh,
        out_type=jax.ShapeDtypeStruct((BP, PD), jnp.float32),
        scratch_types=[
            pltpu.VMEM((b_per_w,), jnp.int32),
            pltpu.VMEM((CHUNK, PD), jnp.float32),
            pltpu.VMEM((CHUNK, PD), jnp.float32),
            pltpu.SemaphoreType.DMA,
            pltpu.SemaphoreType.DMA,
            pltpu.SemaphoreType.DMA,
            pltpu.SemaphoreType.DMA,
        ],
        compiler_params=pltpu.CompilerParams(use_tc_tiling_on_sc=False),
    )
    def gather_kernel(idx_hbm, table_hbm, out_hbm, idx_v, rows0, rows1,
                      g0, g1, o0, o1):
        rows = (rows0, rows1)
        gsem = (g0, g1)
        osem = (o0, o1)
        wid = lax.axis_index("s") * NC + lax.axis_index("c")
        base = wid * b_per_w

        pltpu.sync_copy(idx_hbm.at[pl.ds(base, b_per_w)], idx_v)

        def gather_start(j, b):
            pltpu.async_copy(
                table_hbm.at[idx_v.at[pl.ds(j * CHUNK, CHUNK)]], rows[b],
                gsem[b])

        def gather_wait(b):
            pltpu.make_async_copy(
                table_hbm.at[idx_v.at[pl.ds(0, CHUNK)]], rows[b],
                gsem[b]).wait()

        def out_start(j, b):
            pltpu.async_copy(
                rows[b], out_hbm.at[pl.ds(base + j * CHUNK, CHUNK)], osem[b])

        def out_wait(j, b):
            pltpu.make_async_copy(
                rows[b], out_hbm.at[pl.ds(base + j * CHUNK, CHUNK)],
                osem[b]).wait()

        gather_start(0, 0)

        @pl.loop(0, n_chunks, step=2)
        def pair(j0):
            for b in range(2):
                j = j0 + b
                nb = 1 - b

                @pl.when(jnp.logical_and(j >= 1, j + 1 < n_chunks))
                def _():
                    out_wait(j - 1, nb)

                @pl.when(j + 1 < n_chunks)
                def _():
                    gather_start(j + 1, nb)

                gather_wait(b)
                out_start(j, b)

        out_wait(n_chunks - 2, 0)
        out_wait(n_chunks - 1, 1)

    return gather_kernel


@jax.jit
def kernel(x, weight):
    batch, hist = x.shape
    vocab = weight.shape[0]
    pad_cols = HIST_PAD - hist
    fill = (jax.lax.broadcasted_iota(jnp.int32, (batch, pad_cols), 0)
            * pad_cols
            + jax.lax.broadcasted_iota(jnp.int32, (batch, pad_cols), 1))
    x_pad = jnp.concatenate([x.astype(jnp.int32), fill], axis=1)
    flat_idx = x_pad.reshape(-1)
    table = _make_transpose(vocab)(weight.T)
    out = _make_gather(batch * HIST_PAD)(flat_idx, table)
    return out.reshape(batch, HIST_PAD, PD)[:, :hist, :DIM]


# speedup vs baseline: 1.4470x; 1.4470x over previous
import functools

import jax
import jax.numpy as jnp
from jax import lax
from jax.experimental import pallas as pl
from jax.experimental.pallas import tpu as pltpu
from jax.experimental.pallas import tpu_sc as plsc

DIM = 64
PD = 128
HIST_PAD = 56
NC = 2
NS = 16
NW = NC * NS
CHUNK = 256
VBLK = 8192


def _transpose_body(wt_ref, out_ref):
    out_ref[:, : DIM] = wt_ref[...].T


@functools.cache
def _make_transpose(V: int):
    grid = (V + VBLK - 1) // VBLK
    return pl.pallas_call(
        _transpose_body,
        grid=(grid,),
        in_specs=[pl.BlockSpec((DIM, VBLK), lambda j: (0, j))],
        out_specs=pl.BlockSpec((VBLK, PD), lambda j: (j, 0)),
        out_shape=jax.ShapeDtypeStruct((V, PD), jnp.float32),
    )


@functools.cache
def _make_gather(BP: int):
    b_per_w = BP // NW
    n_chunks = b_per_w // CHUNK
    assert n_chunks % 2 == 0
    mesh = plsc.VectorSubcoreMesh(core_axis_name="c", subcore_axis_name="s")

    @functools.partial(
        pl.kernel,
        mesh=mesh,
        out_type=jax.ShapeDtypeStruct((BP, PD), jnp.float32),
        scratch_types=[
            pltpu.VMEM((b_per_w,), jnp.int32),
            pltpu.VMEM((CHUNK, PD), jnp.float32),
            pltpu.VMEM((CHUNK, PD), jnp.float32),
            pltpu.SemaphoreType.DMA,
            pltpu.SemaphoreType.DMA,
            pltpu.SemaphoreType.DMA,
            pltpu.SemaphoreType.DMA,
        ],
        compiler_params=pltpu.CompilerParams(use_tc_tiling_on_sc=False),
    )
    def gather_kernel(idx_hbm, table_hbm, out_hbm, idx_v, rows0, rows1,
                      g0, g1, o0, o1):
        rows = (rows0, rows1)
        gsem = (g0, g1)
        osem = (o0, o1)
        wid = lax.axis_index("s") * NC + lax.axis_index("c")
        base = wid * b_per_w

        pltpu.sync_copy(idx_hbm.at[pl.ds(base, b_per_w)], idx_v)

        def gather_start(j, b):
            pltpu.async_copy(
                table_hbm.at[idx_v.at[pl.ds(j * CHUNK, CHUNK)]], rows[b],
                gsem[b])

        def gather_wait(b):
            pltpu.make_async_copy(
                table_hbm.at[idx_v.at[pl.ds(0, CHUNK)]], rows[b],
                gsem[b]).wait()

        def out_start(j, b):
            pltpu.async_copy(
                rows[b], out_hbm.at[pl.ds(base + j * CHUNK, CHUNK)], osem[b])

        def out_wait(j, b):
            pltpu.make_async_copy(
                rows[b], out_hbm.at[pl.ds(base + j * CHUNK, CHUNK)],
                osem[b]).wait()

        gather_start(0, 0)

        @pl.loop(0, n_chunks, step=2)
        def pair(j0):
            for b in range(2):
                j = j0 + b
                nb = 1 - b

                @pl.when(jnp.logical_and(j >= 1, j + 1 < n_chunks))
                def _():
                    out_wait(j - 1, nb)

                @pl.when(j + 1 < n_chunks)
                def _():
                    gather_start(j + 1, nb)

                gather_wait(b)
                out_start(j, b)

        out_wait(n_chunks - 2, 0)
        out_wait(n_chunks - 1, 1)

    return gather_kernel


@jax.jit
def kernel(x, weight):
    batch, hist = x.shape
    vocab = weight.shape[0]
    pad_cols = HIST_PAD - hist
    fill = (jax.lax.broadcasted_iota(jnp.int32, (batch, pad_cols), 0)
            * pad_cols
            + jax.lax.broadcasted_iota(jnp.int32, (batch, pad_cols), 1))
    x_pad = jnp.concatenate([x.astype(jnp.int32), fill], axis=1)
    flat_idx = x_pad.reshape(-1)
    table = _make_transpose(vocab)(weight.T)
    out = _make_gather(batch * HIST_PAD)(flat_idx, table)
    return out.reshape(batch, HIST_PAD, PD)[:, :hist, :DIM]


# speedup vs baseline: 1.4703x; 1.0161x over previous
import functools

import jax
import jax.numpy as jnp
from jax import lax
from jax.experimental import pallas as pl
from jax.experimental.pallas import tpu as pltpu
from jax.experimental.pallas import tpu_sc as plsc

DIM = 64
PD = 128
HIST_PAD = 56
NC = 2
NS = 16
NW = NC * NS
CHUNK = 256
VBLK = 16384


def _transpose_body(wt_ref, out_ref):
    out_ref[:, : DIM] = wt_ref[...].T


@functools.cache
def _make_transpose(V: int):
    grid = (V + VBLK - 1) // VBLK
    return pl.pallas_call(
        _transpose_body,
        grid=(grid,),
        in_specs=[pl.BlockSpec((DIM, VBLK), lambda j: (0, j))],
        out_specs=pl.BlockSpec((VBLK, PD), lambda j: (j, 0)),
        out_shape=jax.ShapeDtypeStruct((V, PD), jnp.float32),
    )


@functools.cache
def _make_gather(BP: int):
    b_per_w = BP // NW
    n_chunks = b_per_w // CHUNK
    assert n_chunks % 2 == 0
    mesh = plsc.VectorSubcoreMesh(core_axis_name="c", subcore_axis_name="s")

    @functools.partial(
        pl.kernel,
        mesh=mesh,
        out_type=jax.ShapeDtypeStruct((BP, PD), jnp.float32),
        scratch_types=[
            pltpu.VMEM((b_per_w,), jnp.int32),
            pltpu.VMEM((CHUNK, PD), jnp.float32),
            pltpu.VMEM((CHUNK, PD), jnp.float32),
            pltpu.SemaphoreType.DMA,
            pltpu.SemaphoreType.DMA,
            pltpu.SemaphoreType.DMA,
            pltpu.SemaphoreType.DMA,
        ],
        compiler_params=pltpu.CompilerParams(use_tc_tiling_on_sc=False),
    )
    def gather_kernel(idx_hbm, table_hbm, out_hbm, idx_v, rows0, rows1,
                      g0, g1, o0, o1):
        rows = (rows0, rows1)
        gsem = (g0, g1)
        osem = (o0, o1)
        wid = lax.axis_index("s") * NC + lax.axis_index("c")
        base = wid * b_per_w

        pltpu.sync_copy(idx_hbm.at[pl.ds(base, b_per_w)], idx_v)

        def gather_start(j, b):
            pltpu.async_copy(
                table_hbm.at[idx_v.at[pl.ds(j * CHUNK, CHUNK)]], rows[b],
                gsem[b])

        def gather_wait(b):
            pltpu.make_async_copy(
                table_hbm.at[idx_v.at[pl.ds(0, CHUNK)]], rows[b],
                gsem[b]).wait()

        def out_start(j, b):
            pltpu.async_copy(
                rows[b], out_hbm.at[pl.ds(base + j * CHUNK, CHUNK)], osem[b])

        def out_wait(j, b):
            pltpu.make_async_copy(
                rows[b], out_hbm.at[pl.ds(base + j * CHUNK, CHUNK)],
                osem[b]).wait()

        gather_start(0, 0)

        @pl.loop(0, n_chunks, step=2)
        def pair(j0):
            for b in range(2):
                j = j0 + b
                nb = 1 - b

                @pl.when(jnp.logical_and(j >= 1, j + 1 < n_chunks))
                def _():
                    out_wait(j - 1, nb)

                @pl.when(j + 1 < n_chunks)
                def _():
                    gather_start(j + 1, nb)

                gather_wait(b)
                out_start(j, b)

        out_wait(n_chunks - 2, 0)
        out_wait(n_chunks - 1, 1)

    return gather_kernel


@jax.jit
def kernel(x, weight):
    batch, hist = x.shape
    vocab = weight.shape[0]
    pad_cols = HIST_PAD - hist
    fill = (jax.lax.broadcasted_iota(jnp.int32, (batch, pad_cols), 0)
            * pad_cols
            + jax.lax.broadcasted_iota(jnp.int32, (batch, pad_cols), 1))
    x_pad = jnp.concatenate([x.astype(jnp.int32), fill], axis=1)
    flat_idx = x_pad.reshape(-1)
    table = _make_transpose(vocab)(weight.T)
    out = _make_gather(batch * HIST_PAD)(flat_idx, table)
    return out.reshape(batch, HIST_PAD, PD)[:, :hist, :DIM]


# speedup vs baseline: 1.4826x; 1.0084x over previous
import functools

import jax
import jax.numpy as jnp
from jax import lax
from jax.experimental import pallas as pl
from jax.experimental.pallas import tpu as pltpu
from jax.experimental.pallas import tpu_sc as plsc

DIM = 64
PD = 128
HIST_PAD = 56
NC = 2
NS = 16
NW = NC * NS
CHUNK = 256
VBLK = 32768


def _transpose_body(wt_ref, out_ref):
    out_ref[:, : DIM] = wt_ref[...].T


@functools.cache
def _make_transpose(V: int):
    grid = (V + VBLK - 1) // VBLK
    return pl.pallas_call(
        _transpose_body,
        grid=(grid,),
        in_specs=[pl.BlockSpec((DIM, VBLK), lambda j: (0, j))],
        out_specs=pl.BlockSpec((VBLK, PD), lambda j: (j, 0)),
        out_shape=jax.ShapeDtypeStruct((V, PD), jnp.float32),
    )


@functools.cache
def _make_gather(BP: int):
    b_per_w = BP // NW
    n_chunks = b_per_w // CHUNK
    assert n_chunks % 2 == 0
    mesh = plsc.VectorSubcoreMesh(core_axis_name="c", subcore_axis_name="s")

    @functools.partial(
        pl.kernel,
        mesh=mesh,
        out_type=jax.ShapeDtypeStruct((BP, PD), jnp.float32),
        scratch_types=[
            pltpu.VMEM((b_per_w,), jnp.int32),
            pltpu.VMEM((CHUNK, PD), jnp.float32),
            pltpu.VMEM((CHUNK, PD), jnp.float32),
            pltpu.SemaphoreType.DMA,
            pltpu.SemaphoreType.DMA,
            pltpu.SemaphoreType.DMA,
            pltpu.SemaphoreType.DMA,
        ],
        compiler_params=pltpu.CompilerParams(use_tc_tiling_on_sc=False),
    )
    def gather_kernel(idx_hbm, table_hbm, out_hbm, idx_v, rows0, rows1,
                      g0, g1, o0, o1):
        rows = (rows0, rows1)
        gsem = (g0, g1)
        osem = (o0, o1)
        wid = lax.axis_index("s") * NC + lax.axis_index("c")
        base = wid * b_per_w

        pltpu.sync_copy(idx_hbm.at[pl.ds(base, b_per_w)], idx_v)

        def gather_start(j, b):
            pltpu.async_copy(
                table_hbm.at[idx_v.at[pl.ds(j * CHUNK, CHUNK)]], rows[b],
                gsem[b])

        def gather_wait(b):
            pltpu.make_async_copy(
                table_hbm.at[idx_v.at[pl.ds(0, CHUNK)]], rows[b],
                gsem[b]).wait()

        def out_start(j, b):
            pltpu.async_copy(
                rows[b], out_hbm.at[pl.ds(base + j * CHUNK, CHUNK)], osem[b])

        def out_wait(j, b):
            pltpu.make_async_copy(
                rows[b], out_hbm.at[pl.ds(base + j * CHUNK, CHUNK)],
                osem[b]).wait()

        gather_start(0, 0)

        @pl.loop(0, n_chunks, step=2)
        def pair(j0):
            for b in range(2):
                j = j0 + b
                nb = 1 - b

                @pl.when(jnp.logical_and(j >= 1, j + 1 < n_chunks))
                def _():
                    out_wait(j - 1, nb)

                @pl.when(j + 1 < n_chunks)
                def _():
                    gather_start(j + 1, nb)

                gather_wait(b)
                out_start(j, b)

        out_wait(n_chunks - 2, 0)
        out_wait(n_chunks - 1, 1)

    return gather_kernel


@jax.jit
def kernel(x, weight):
    batch, hist = x.shape
    vocab = weight.shape[0]
    pad_cols = HIST_PAD - hist
    fill = (jax.lax.broadcasted_iota(jnp.int32, (batch, pad_cols), 0)
            * pad_cols
            + jax.lax.broadcasted_iota(jnp.int32, (batch, pad_cols), 1))
    x_pad = jnp.concatenate([x.astype(jnp.int32), fill], axis=1)
    flat_idx = x_pad.reshape(-1)
    table = _make_transpose(vocab)(weight.T)
    out = _make_gather(batch * HIST_PAD)(flat_idx, table)
    return out.reshape(batch, HIST_PAD, PD)[:, :hist, :DIM]


# speedup vs baseline: 1.5448x; 1.0420x over previous
import functools

import jax
import jax.numpy as jnp
from jax import lax
from jax.experimental import pallas as pl
from jax.experimental.pallas import tpu as pltpu
from jax.experimental.pallas import tpu_sc as plsc

DIM = 64
PD = 128
HIST = 50
HIST_PAD = 56
NC = 2
NS = 16
NW = NC * NS
BCHUNK = 4
VBLK = 32768


def _transpose_body(wt_ref, out_ref):
    out_ref[:, : DIM] = wt_ref[...].T


@functools.cache
def _make_transpose(V: int):
    grid = (V + VBLK - 1) // VBLK
    return pl.pallas_call(
        _transpose_body,
        grid=(grid,),
        in_specs=[pl.BlockSpec((DIM, VBLK), lambda j: (0, j))],
        out_specs=pl.BlockSpec((VBLK, PD), lambda j: (j, 0)),
        out_shape=jax.ShapeDtypeStruct((V, PD), jnp.float32),
    )


@functools.cache
def _make_gather(B: int):
    b_per_w = B // NW
    batch_per_w = b_per_w // HIST
    n_chunks = batch_per_w // BCHUNK
    assert n_chunks % 2 == 0
    ICHUNK = BCHUNK * HIST
    mesh = plsc.VectorSubcoreMesh(core_axis_name="c", subcore_axis_name="s")

    @functools.partial(
        pl.kernel,
        mesh=mesh,
        out_type=jax.ShapeDtypeStruct((B // HIST * HIST_PAD, PD),
                                      jnp.float32),
        scratch_types=[
            pltpu.VMEM((b_per_w,), jnp.int32),
            pltpu.VMEM((ICHUNK, PD), jnp.float32),
            pltpu.VMEM((ICHUNK, PD), jnp.float32),
            pltpu.SemaphoreType.DMA,
            pltpu.SemaphoreType.DMA,
            pltpu.SemaphoreType.DMA,
            pltpu.SemaphoreType.DMA,
        ],
        compiler_params=pltpu.CompilerParams(use_tc_tiling_on_sc=False),
    )
    def gather_kernel(idx_hbm, table_hbm, out_hbm, idx_v, rows0, rows1,
                      g0, g1, o0, o1):
        rows = (rows0, rows1)
        gsem = (g0, g1)
        osem = (o0, o1)
        wid = lax.axis_index("s") * NC + lax.axis_index("c")
        base = wid * b_per_w
        bbase = wid * batch_per_w

        pltpu.sync_copy(idx_hbm.at[pl.ds(base, b_per_w)], idx_v)

        def gather_start(j, b):
            pltpu.async_copy(
                table_hbm.at[idx_v.at[pl.ds(j * ICHUNK, ICHUNK)]], rows[b],
                gsem[b])

        def gather_wait(b):
            pltpu.make_async_copy(
                table_hbm.at[idx_v.at[pl.ds(0, ICHUNK)]], rows[b],
                gsem[b]).wait()

        def out_start(j, b):
            for i in range(BCHUNK):
                pltpu.async_copy(
                    rows[b].at[pl.ds(i * HIST, HIST)],
                    out_hbm.at[pl.ds((bbase + j * BCHUNK + i) * HIST_PAD,
                                     HIST)],
                    osem[b])

        def out_wait(j, b):
            for i in range(BCHUNK):
                pltpu.make_async_copy(
                    rows[b].at[pl.ds(i * HIST, HIST)],
                    out_hbm.at[pl.ds((bbase + j * BCHUNK + i) * HIST_PAD,
                                     HIST)],
                    osem[b]).wait()

        gather_start(0, 0)

        @pl.loop(0, n_chunks, step=2)
        def pair(j0):
            for b in range(2):
                j = j0 + b
                nb = 1 - b

                @pl.when(jnp.logical_and(j >= 1, j + 1 < n_chunks))
                def _():
                    out_wait(j - 1, nb)

                @pl.when(j + 1 < n_chunks)
                def _():
                    gather_start(j + 1, nb)

                gather_wait(b)
                out_start(j, b)

        out_wait(n_chunks - 2, 0)
        out_wait(n_chunks - 1, 1)

    return gather_kernel


@jax.jit
def kernel(x, weight):
    batch, hist = x.shape
    vocab = weight.shape[0]
    flat_idx = x.reshape(-1).astype(jnp.int32)
    table = _make_transpose(vocab)(weight.T)
    out = _make_gather(batch * hist)(flat_idx, table)
    return out.reshape(batch, HIST_PAD, PD)[:, :hist, :DIM]
